# baseline (speedup 1.0000x reference)
import functools

import jax
import jax.numpy as jnp
from jax import lax
from jax.experimental import pallas as pl
from jax.experimental.pallas import tpu as pltpu
from jax.experimental.pallas import tpu_sc as plsc

_BLOCK = 64
_NSWAP = 256
_NC = 2
_NS = 16
_NW = _NC * _NS
_LANES = 16
_NBUF = 2
_NBUF_S = 2


def _sc_swap_body(cache_hbm, srcs_hbm, dsts_hbm, out_hbm,
                  sv, dv, tbuf, spbuf, sems):
    nper = cache_hbm.shape[0] // _NW
    cid = lax.axis_index("c")
    sid = lax.axis_index("s")
    wid = sid * _NC + cid
    base = wid * nper

    pltpu.sync_copy(srcs_hbm, sv)
    pltpu.sync_copy(dsts_hbm, dv)

    iota = lax.iota(jnp.int32, _LANES)
    bid0 = iota + base
    bid1 = iota + (base + _LANES)

    def scan_chunk(c, carry, flip):
        m0, m1 = carry
        svec = sv[pl.ds(c * _LANES, _LANES)]
        dvec = dv[pl.ds(c * _LANES, _LANES)]
        if flip:
            svec, dvec = dvec, svec
        for j in range(_LANES):
            d = dvec[j]
            s = svec[j]
            m0 = jnp.where(bid0 == d, s, m0)
            m1 = jnp.where(bid1 == d, s, m1)
        return m0, m1

    nchunk = _NSWAP // _LANES
    m0, m1 = lax.fori_loop(
        0, nchunk, functools.partial(scan_chunk, flip=False), (bid0, bid1))
    m0, m1 = lax.fori_loop(
        0, nchunk, functools.partial(scan_chunk, flip=True), (m0, m1))
    ms = (m0, m1)

    def blkidx(b):
        return ms[b // _LANES][b % _LANES]

    def make_pipe(items, buf_at, src_at, dst_at, gsem, ssem, nbuf):
        def gstart(i, slot):
            pltpu.make_async_copy(
                src_at(cache_hbm, items[i]), buf_at(slot), gsem.at[slot]
            ).start()

        def gwait(slot):
            pltpu.make_async_copy(
                src_at(cache_hbm, items[0]), buf_at(slot), gsem.at[slot]
            ).wait()

        def sstart(i, slot):
            pltpu.make_async_copy(
                buf_at(slot), dst_at(out_hbm, items[i]), ssem.at[slot]
            ).start()

        def swait(slot):
            pltpu.make_async_copy(
                buf_at(slot), dst_at(out_hbm, items[0]), ssem.at[slot]
            ).wait()

        return dict(n=len(items), nbuf=nbuf, gstart=gstart, gwait=gwait,
                    sstart=sstart, swait=swait)

    def pipe_prime(p):
        for t in range(p["nbuf"] - 1):
            p["gstart"](t, t)

    def pipe_flush(p, k):
        slot = k % p["nbuf"]
        p["gwait"](slot)
        p["sstart"](k, slot)

    def pipe_refill(p, k):
        nxt = k + p["nbuf"] - 1
        if nxt < p["n"]:
            if k >= 1:
                p["swait"](nxt % p["nbuf"])
            p["gstart"](nxt, nxt % p["nbuf"])

    def pipe_drain(p):
        for t in range(p["nbuf"]):
            p["swait"]((p["n"] - 1 - t) % p["nbuf"])

    halfw = cache_hbm.shape[1] // 2
    t_items = [(blkidx(b), base + b) for b in range(0, nper, 2)]
    s_items = [(blkidx(b), base + b, h) for b in range(1, nper, 2)
               for h in (0, 1)]

    pipe_t = make_pipe(t_items,
                       lambda slot: tbuf.at[slot],
                       lambda ref, it: ref.at[it[0]],
                       lambda ref, it: ref.at[it[1]],
                       sems[0][0], sems[0][1], _NBUF)
    pipe_s = make_pipe(s_items,
                       lambda slot: spbuf.at[sid].at[slot],
                       lambda ref, it: ref.at[it[0], pl.ds(it[2] * halfw, halfw)],
                       lambda ref, it: ref.at[it[1], pl.ds(it[2] * halfw, halfw)],
                       sems[1][0], sems[1][1], _NBUF_S)

    pipe_prime(pipe_t)
    pipe_prime(pipe_s)
    for k in range(len(t_items)):
        pipe_flush(pipe_t, k)
        pipe_flush(pipe_s, 2 * k)
        pipe_refill(pipe_s, 2 * k)
        pipe_refill(pipe_t, k)
        pipe_flush(pipe_s, 2 * k + 1)
        pipe_refill(pipe_s, 2 * k + 1)
    pipe_drain(pipe_t)
    pipe_drain(pipe_s)


def kernel(cache, srcs, dsts, block_size):
    rows, d = cache.shape
    nb = rows // _BLOCK
    blk_w = _BLOCK * d
    cache2 = cache.reshape(nb, blk_w)

    def body(cache_hbm, srcs_hbm, dsts_hbm, out_hbm, sv, dv, tbuf, spbuf,
             g0, s0, g1, s1):
        _sc_swap_body(cache_hbm, srcs_hbm, dsts_hbm, out_hbm,
                      sv, dv, tbuf, spbuf, ((g0, s0), (g1, s1)))

    run = functools.partial(
        pl.kernel,
        out_type=jax.ShapeDtypeStruct((nb, blk_w), cache.dtype),
        mesh=plsc.VectorSubcoreMesh(core_axis_name="c", subcore_axis_name="s"),
        scratch_types=[
            pltpu.VMEM((_NSWAP,), jnp.int32),
            pltpu.VMEM((_NSWAP,), jnp.int32),
            pltpu.VMEM((_NBUF, blk_w), cache.dtype),
            pltpu.VMEM_SHARED((_NS, _NBUF_S, blk_w // 2), cache.dtype),
            pltpu.SemaphoreType.DMA((_NBUF,)),
            pltpu.SemaphoreType.DMA((_NBUF,)),
            pltpu.SemaphoreType.DMA((_NBUF,)),
            pltpu.SemaphoreType.DMA((_NBUF,)),
        ],
    )(body)
    return run(cache2, srcs, dsts).reshape(rows, d)


# speedup vs baseline: 3.8347x; 3.8347x over previous
import functools

import jax
import jax.numpy as jnp
from jax import lax
from jax.experimental import pallas as pl
from jax.experimental.pallas import tpu as pltpu
from jax.experimental.pallas import tpu_sc as plsc

_BLOCK = 64
_NSWAP = 256
_NC = 2
_NS = 16
_NW = _NC * _NS
_LANES = 16
_NBUF = 2
_NBUF_S = 2


def _sc_swap_body(cache_hbm, srcs_hbm, dsts_hbm, out_hbm,
                  sv, dv, tbuf, spbuf, sems):
    nper = cache_hbm.shape[0] // _BLOCK // _NW
    cid = lax.axis_index("c")
    sid = lax.axis_index("s")
    wid = sid * _NC + cid
    base = wid * nper

    pltpu.sync_copy(srcs_hbm, sv)
    pltpu.sync_copy(dsts_hbm, dv)

    iota = lax.iota(jnp.int32, _LANES)
    bid0 = iota + base
    bid1 = iota + (base + _LANES)

    def scan_chunk(c, carry, flip):
        m0, m1 = carry
        svec = sv[pl.ds(c * _LANES, _LANES)]
        dvec = dv[pl.ds(c * _LANES, _LANES)]
        if flip:
            svec, dvec = dvec, svec
        for j in range(_LANES):
            d = dvec[j]
            s = svec[j]
            m0 = jnp.where(bid0 == d, s, m0)
            m1 = jnp.where(bid1 == d, s, m1)
        return m0, m1

    nchunk = _NSWAP // _LANES
    m0, m1 = lax.fori_loop(
        0, nchunk, functools.partial(scan_chunk, flip=False), (bid0, bid1))
    m0, m1 = lax.fori_loop(
        0, nchunk, functools.partial(scan_chunk, flip=True), (m0, m1))
    ms = (m0, m1)

    def blkidx(b):
        return ms[b // _LANES][b % _LANES]

    def make_pipe(items, buf_at, nrows, gsem, ssem, nbuf):
        def gstart(i, slot):
            src, dst = items[i]
            pltpu.make_async_copy(
                cache_hbm.at[pl.ds(src, nrows)], buf_at(slot), gsem.at[slot]
            ).start()

        def gwait(slot):
            pltpu.make_async_copy(
                cache_hbm.at[pl.ds(0, nrows)], buf_at(slot), gsem.at[slot]
            ).wait()

        def sstart(i, slot):
            src, dst = items[i]
            pltpu.make_async_copy(
                buf_at(slot), out_hbm.at[pl.ds(dst, nrows)], ssem.at[slot]
            ).start()

        def swait(slot):
            pltpu.make_async_copy(
                buf_at(slot), out_hbm.at[pl.ds(0, nrows)], ssem.at[slot]
            ).wait()

        return dict(n=len(items), nbuf=nbuf, gstart=gstart, gwait=gwait,
                    sstart=sstart, swait=swait)

    def pipe_prime(p):
        for t in range(p["nbuf"] - 1):
            p["gstart"](t, t)

    def pipe_flush(p, k):
        slot = k % p["nbuf"]
        p["gwait"](slot)
        p["sstart"](k, slot)

    def pipe_refill(p, k):
        nxt = k + p["nbuf"] - 1
        if nxt < p["n"]:
            if k >= 1:
                p["swait"](nxt % p["nbuf"])
            p["gstart"](nxt, nxt % p["nbuf"])

    def pipe_drain(p):
        for t in range(p["nbuf"]):
            p["swait"]((p["n"] - 1 - t) % p["nbuf"])

    half = _BLOCK // 2
    t_items = [(blkidx(b) * _BLOCK, (base + b) * _BLOCK)
               for b in range(0, nper, 2)]
    s_items = [(blkidx(b) * _BLOCK + h * half, (base + b) * _BLOCK + h * half)
               for b in range(1, nper, 2) for h in (0, 1)]

    pipe_t = make_pipe(t_items, lambda slot: tbuf.at[slot], _BLOCK,
                       sems[0][0], sems[0][1], _NBUF)
    pipe_s = make_pipe(s_items, lambda slot: spbuf.at[sid].at[slot], half,
                       sems[1][0], sems[1][1], _NBUF_S)

    pipe_prime(pipe_t)
    pipe_prime(pipe_s)
    for k in range(len(t_items)):
        pipe_flush(pipe_t, k)
        pipe_flush(pipe_s, 2 * k)
        pipe_refill(pipe_s, 2 * k)
        pipe_refill(pipe_t, k)
        pipe_flush(pipe_s, 2 * k + 1)
        pipe_refill(pipe_s, 2 * k + 1)
    pipe_drain(pipe_t)
    pipe_drain(pipe_s)


def kernel(cache, srcs, dsts, block_size):
    rows, d = cache.shape

    def body(cache_hbm, srcs_hbm, dsts_hbm, out_hbm, sv, dv, tbuf, spbuf,
             g0, s0, g1, s1):
        _sc_swap_body(cache_hbm, srcs_hbm, dsts_hbm, out_hbm,
                      sv, dv, tbuf, spbuf, ((g0, s0), (g1, s1)))

    run = functools.partial(
        pl.kernel,
        out_type=jax.ShapeDtypeStruct((rows, d), cache.dtype),
        mesh=plsc.VectorSubcoreMesh(core_axis_name="c", subcore_axis_name="s"),
        scratch_types=[
            pltpu.VMEM((_NSWAP,), jnp.int32),
            pltpu.VMEM((_NSWAP,), jnp.int32),
            pltpu.VMEM((_NBUF, _BLOCK, d), cache.dtype),
            pltpu.VMEM_SHARED((_NS, _NBUF_S, _BLOCK // 2, d), cache.dtype),
            pltpu.SemaphoreType.DMA((_NBUF,)),
            pltpu.SemaphoreType.DMA((_NBUF,)),
            pltpu.SemaphoreType.DMA((_NBUF,)),
            pltpu.SemaphoreType.DMA((_NBUF,)),
        ],
    )(body)
    return run(cache, srcs, dsts)
